# baseline (speedup 1.0000x reference)
import jax
import jax.numpy as jnp
from jax.experimental import pallas as pl
from jax.experimental.pallas import tpu as pltpu

_NEG_SLOPE = 0.01


def _leaky(v):
    return jnp.where(v > 0, v, v * _NEG_SLOPE)


def _to_fmajor(r, tm):
    return r.reshape(tm, 4, 32).transpose(2, 0, 1).reshape(32, tm * 4)


def _mixhop_body(x_ref, adj_ref, w_ref, b_ref, o_ref, z_ref, u_ref, pu_ref):
    ph = pl.program_id(1)
    i = pl.program_id(2)
    tm = adj_ref.shape[1]

    @pl.when(ph == 0)
    def _transform():
        xb = x_ref[0]
        h0 = jax.lax.dot_general(w_ref[:, 0:32], xb, (((0,), (0,)), ((), ())),
                                 preferred_element_type=jnp.float32)
        o_ref[0, 0] = _leaky(h0 + b_ref[0, 0:32][:, None])
        h = jax.lax.dot_general(xb, w_ref[:, 32:96], (((0,), (0,)), ((), ())),
                                preferred_element_type=jnp.float32)
        h = h + b_ref[0, 32:96][None, :]
        h = h.reshape(tm, 4, 64)
        z_ref[pl.ds(i * tm, tm), :] = h[:, :, 0:32].reshape(tm, 128)
        u_ref[pl.ds(i * tm, tm), :] = h[:, :, 32:64].reshape(tm, 128)

    @pl.when(ph == 1)
    def _hop1():
        a = adj_ref[0].astype(jnp.bfloat16)
        z = z_ref[...].astype(jnp.bfloat16)
        u = u_ref[...].astype(jnp.bfloat16)
        o_ref[0, 0] = _leaky(_to_fmajor(
            jnp.dot(a, z, preferred_element_type=jnp.float32), tm))
        pu_ref[pl.ds(i * tm, tm), :] = jnp.dot(
            a, u, preferred_element_type=jnp.float32)

    @pl.when(ph == 2)
    def _hop2():
        a = adj_ref[0].astype(jnp.bfloat16)
        pu = pu_ref[...].astype(jnp.bfloat16)
        o_ref[0, 0] = _leaky(_to_fmajor(
            jnp.dot(a, pu, preferred_element_type=jnp.float32), tm))


def kernel(x, adj, W0, b0, W1, b1, W2, b2):
    B, F_in, N, T = x.shape
    F_out = W0.shape[1]
    C = T * F_out
    Tm = 512

    xf = x.reshape(B, F_in, N * T)
    Wall = jnp.concatenate([W0, W1, W2], axis=1)
    ball = jnp.concatenate([b0, b1, b2]).reshape(1, 3 * F_out)

    stacked = pl.pallas_call(
        _mixhop_body,
        grid=(B, 3, N // Tm),
        in_specs=[
            pl.BlockSpec((1, F_in, Tm * T),
                         lambda b, ph, i: (b, 0, jnp.where(ph == 0, i, 0))),
            pl.BlockSpec((1, Tm, N),
                         lambda b, ph, i: (b, jnp.where(ph == 0, 0, i), 0)),
            pl.BlockSpec((F_in, 3 * F_out), lambda b, ph, i: (0, 0)),
            pl.BlockSpec((1, 3 * F_out), lambda b, ph, i: (0, 0)),
        ],
        out_specs=pl.BlockSpec((1, 1, F_out, Tm * T),
                               lambda b, ph, i: (b, ph, 0, i)),
        out_shape=jax.ShapeDtypeStruct((B, 3, F_out, N * T), jnp.float32),
        scratch_shapes=[
            pltpu.VMEM((N, C), jnp.float32),
            pltpu.VMEM((N, C), jnp.float32),
            pltpu.VMEM((N, C), jnp.float32),
        ],
    )(xf, adj, Wall, ball)

    return stacked.reshape(B, 3 * F_out, N, T)


# speedup vs baseline: 2.0827x; 2.0827x over previous
import jax
import jax.numpy as jnp
from jax.experimental import pallas as pl
from jax.experimental.pallas import tpu as pltpu

_NEG_SLOPE = 0.01
_TM = 512


def _leaky(v):
    return jnp.where(v > 0, v, v * _NEG_SLOPE)


def _mixhop_body(x_ref, adj_ref, w_ref, b_ref, o_ref, z_ref, u_ref, pu_ref):
    ph = pl.program_id(1)
    i = pl.program_id(2)
    tm = adj_ref.shape[1]

    @pl.when(ph == 0)
    def _transform():
        xb = x_ref[0]
        h = jax.lax.dot_general(xb, w_ref[...], (((0,), (0,)), ((), ())),
                                preferred_element_type=jnp.float32)
        h = h + b_ref[0][None, :]
        h = h.reshape(tm, 4, 96)
        o_ref[0, 0] = h[:, :, 0:32].reshape(tm, 128)
        z_ref[pl.ds(i * tm, tm), :] = h[:, :, 32:64].reshape(tm, 128)
        u_ref[pl.ds(i * tm, tm), :] = h[:, :, 64:96].reshape(tm, 128)

    @pl.when(ph == 1)
    def _hop1():
        a = adj_ref[0].astype(jnp.bfloat16)
        z = z_ref[...].astype(jnp.bfloat16)
        u = u_ref[...].astype(jnp.bfloat16)
        o_ref[0, 0] = jnp.dot(a, z, preferred_element_type=jnp.float32)
        pu_ref[pl.ds(i * tm, tm), :] = jnp.dot(
            a, u, preferred_element_type=jnp.float32)

    @pl.when(ph == 2)
    def _hop2():
        a = adj_ref[0].astype(jnp.bfloat16)
        pu = pu_ref[...].astype(jnp.bfloat16)
        o_ref[0, 0] = jnp.dot(a, pu, preferred_element_type=jnp.float32)


def kernel(x, adj, W0, b0, W1, b1, W2, b2):
    B, F_in, N, T = x.shape
    F_out = W0.shape[1]
    C = T * F_out
    Tm = _TM

    xf = x.reshape(B, F_in, N * T)
    Wall = jnp.concatenate([W0, W1, W2], axis=1)
    ball = jnp.concatenate([b0, b1, b2]).reshape(1, 3 * F_out)

    stacked = pl.pallas_call(
        _mixhop_body,
        grid=(B, 3, N // Tm),
        in_specs=[
            pl.BlockSpec((1, F_in, Tm * T),
                         lambda b, ph, i: (b, 0, jnp.where(ph == 0, i, 0))),
            pl.BlockSpec((1, Tm, N),
                         lambda b, ph, i: (b, jnp.where(ph == 0, 0, i), 0)),
            pl.BlockSpec((F_in, 3 * F_out), lambda b, ph, i: (0, 0)),
            pl.BlockSpec((1, 3 * F_out), lambda b, ph, i: (0, 0)),
        ],
        out_specs=pl.BlockSpec((1, 1, Tm, C), lambda b, ph, i: (b, ph, i, 0)),
        out_shape=jax.ShapeDtypeStruct((B, 3, N, C), jnp.float32),
        scratch_shapes=[
            pltpu.VMEM((N, C), jnp.float32),
            pltpu.VMEM((N, C), jnp.float32),
            pltpu.VMEM((N, C), jnp.float32),
        ],
    )(xf, adj, Wall, ball)

    out = _leaky(stacked.reshape(B, 3, N, T, F_out).transpose(0, 1, 4, 2, 3))
    return out.reshape(B, 3 * F_out, N, T)


# speedup vs baseline: 2.2667x; 1.0884x over previous
import jax
import jax.numpy as jnp
from jax.experimental import pallas as pl
from jax.experimental.pallas import tpu as pltpu

_NEG_SLOPE = 0.01
_TM = 1024


def _leaky(v):
    return jnp.where(v > 0, v, v * _NEG_SLOPE)


def _mixhop_body(x_ref, adj_ref, w_ref, b_ref, o_ref, z_ref, u_ref, pu_ref):
    ph = pl.program_id(1)
    i = pl.program_id(2)
    tm = adj_ref.shape[1]

    @pl.when(ph == 0)
    def _transform():
        xb = x_ref[0]
        h = jax.lax.dot_general(xb, w_ref[...], (((0,), (0,)), ((), ())),
                                preferred_element_type=jnp.float32)
        h = h + b_ref[0][None, :]
        h = h.reshape(tm, 4, 96)
        o_ref[0, 0] = _leaky(h[:, :, 0:32].reshape(tm, 128))
        z_ref[pl.ds(i * tm, tm), :] = h[:, :, 32:64].reshape(tm, 128)
        u_ref[pl.ds(i * tm, tm), :] = h[:, :, 64:96].reshape(tm, 128)

    @pl.when(ph == 1)
    def _hop1():
        a = adj_ref[0].astype(jnp.bfloat16)
        z = z_ref[...].astype(jnp.bfloat16)
        u = u_ref[...].astype(jnp.bfloat16)
        o_ref[0, 0] = _leaky(jnp.dot(a, z, preferred_element_type=jnp.float32))
        pu_ref[pl.ds(i * tm, tm), :] = jnp.dot(
            a, u, preferred_element_type=jnp.float32)

    @pl.when(ph == 2)
    def _hop2():
        a = adj_ref[0].astype(jnp.bfloat16)
        pu = pu_ref[...].astype(jnp.bfloat16)
        o_ref[0, 0] = _leaky(jnp.dot(a, pu, preferred_element_type=jnp.float32))


def kernel(x, adj, W0, b0, W1, b1, W2, b2):
    B, F_in, N, T = x.shape
    F_out = W0.shape[1]
    C = T * F_out
    Tm = _TM

    xf = x.reshape(B, F_in, N * T)
    Wall = jnp.concatenate([W0, W1, W2], axis=1)
    ball = jnp.concatenate([b0, b1, b2]).reshape(1, 3 * F_out)

    stacked = pl.pallas_call(
        _mixhop_body,
        grid=(B, 3, N // Tm),
        in_specs=[
            pl.BlockSpec((1, F_in, Tm * T),
                         lambda b, ph, i: (b, 0, jnp.where(ph == 0, i, 0))),
            pl.BlockSpec((1, Tm, N),
                         lambda b, ph, i: (b, jnp.where(ph == 0, 0, i), 0)),
            pl.BlockSpec((F_in, 3 * F_out), lambda b, ph, i: (0, 0)),
            pl.BlockSpec((1, 3 * F_out), lambda b, ph, i: (0, 0)),
        ],
        out_specs=pl.BlockSpec((1, 1, Tm, C), lambda b, ph, i: (b, ph, i, 0)),
        out_shape=jax.ShapeDtypeStruct((B, 3, N, C), jnp.float32),
        scratch_shapes=[
            pltpu.VMEM((N, C), jnp.float32),
            pltpu.VMEM((N, C), jnp.float32),
            pltpu.VMEM((N, C), jnp.float32),
        ],
    )(xf, adj, Wall, ball)

    out = stacked.reshape(B, 3, N, T, F_out).transpose(0, 1, 4, 2, 3)
    return out.reshape(B, 3 * F_out, N, T)
